# Initial kernel scaffold; baseline (speedup 1.0000x reference)
import jax
import jax.numpy as jnp
from jax.experimental import pallas as pl


def kernel(edge_index, edge_label_index, embedding, alpha):
    raise NotImplementedError("write your pallas kernel here")


# speedup vs baseline: 9.0411x; 9.0411x over previous
import functools

import jax
import jax.numpy as jnp
from jax import lax
from jax.experimental import pallas as pl
from jax.experimental.pallas import tpu as pltpu
from jax.experimental.pallas import tpu_sc as plsc

N = 50000
D = 64
E = 800000
NL = 100000
NC, NS = 2, 16

NHALF = 25600
NPAD = 2 * NHALF
TR = NHALF // NS
EPT = E // NS
CB = 80
EB = 2000
NEB = EPT // EB
NCB = EB // CB
NLP = 102400
LPT = NLP // (NC * NS)
LCB = 128
LNCH = LPT // LCB

_MESH = plsc.VectorSubcoreMesh(
    core_axis_name="c", subcore_axis_name="s", num_cores=NC, num_subcores=NS
)

_f32 = jnp.float32
_i32 = jnp.int32

_CP = pltpu.CompilerParams(
    use_tc_tiling_on_sc=False, needs_layout_passes=False
)


def _rsqrt16(x):
    s = jnp.maximum(x, 1.0)
    for _ in range(18):
        s = 0.5 * (s + x / s)
    return jnp.where(x > 0.0, 1.0 / s, 0.0)


def _zero_rows(buf, rows):
    z = jnp.zeros((16,), _f32)

    def body(r, _):
        for v in range(D // 16):
            buf[r, pl.ds(16 * v, 16)] = z
        return _

    lax.fori_loop(0, rows, body, None)


@functools.partial(
    pl.kernel,
    out_type=(
        jax.ShapeDtypeStruct((NPAD,), _f32),
        jax.ShapeDtypeStruct((NPAD, D), _f32),
        jax.ShapeDtypeStruct((NPAD, D), _f32),
    ),
    mesh=_MESH,
    compiler_params=_CP,
    scratch_types=(
        pltpu.VMEM_SHARED((NPAD,), _f32),
        pltpu.VMEM((NPAD // NS,), _f32),
        pltpu.VMEM((CB,), _f32),
        pltpu.VMEM((10000,), _i32),
        pltpu.VMEM((NPAD // NS,), _f32),
        pltpu.VMEM((CB,), _f32),
        pltpu.VMEM((CB, D), _f32),
        pltpu.VMEM((CB, D), _f32),
        pltpu.VMEM((CB, D), _f32),
        pltpu.VMEM((16,), _f32),
    ),
)
def _k0(col_hbm, emb_hbm, w_hbm, dinv_o, xt0_o, out0_o,
        deg_sh, zb, ones, colv, degv, dv, embv, xtv, ov, wv):
    c = lax.axis_index("c")
    s = lax.axis_index("s")
    spt = NPAD // NS

    def zb_body(i, _):
        zb[pl.ds(16 * i, 16)] = jnp.zeros((16,), _f32)
        return _
    lax.fori_loop(0, spt // 16, zb_body, None)
    pltpu.sync_copy(zb.at[pl.ds(0, spt)], deg_sh.at[pl.ds(s * spt, spt)])
    for k in range(CB // 16):
        ones[pl.ds(16 * k, 16)] = jnp.ones((16,), _f32)
    pltpu.sync_copy(w_hbm, wv)
    plsc.subcore_barrier()

    for b in range(5):
        pltpu.sync_copy(col_hbm.at[pl.ds(s * EPT + b * 10000, 10000)], colv)

        def deg_body(j, _):
            pltpu.sync_copy(
                ones, deg_sh.at[colv.at[pl.ds(j * CB, CB)]], add=True
            )
            return _
        lax.fori_loop(0, 10000 // CB, deg_body, None)
    plsc.subcore_barrier()

    @pl.when(c == 0)
    def _():
        pltpu.sync_copy(deg_sh.at[pl.ds(s * spt, spt)], degv)

        def nwt(i, _):
            degv[pl.ds(16 * i, 16)] = _rsqrt16(degv[pl.ds(16 * i, 16)])
            return _
        lax.fori_loop(0, spt // 16, nwt, None)
        pltpu.sync_copy(degv, dinv_o.at[pl.ds(s * spt, spt)])

    g0 = c * NHALF + s * TR

    def x_body(ch, _):
        gb = g0 + ch * CB
        eb = jnp.minimum(gb, N - CB)
        pltpu.sync_copy(emb_hbm.at[pl.ds(eb, CB)], embv)
        pltpu.sync_copy(deg_sh.at[pl.ds(gb, CB)], dv)
        for k in range(CB // 16):
            dv[pl.ds(16 * k, 16)] = _rsqrt16(dv[pl.ds(16 * k, 16)])
        w0 = wv[pl.ds(0, 16)][0]

        def row_grp(g, _):
            dvec = dv[pl.ds(16 * g, 16)]
            for k in range(16):
                d = dvec[k]
                r = 16 * g + k
                for v in range(D // 16):
                    e = embv[r, pl.ds(16 * v, 16)]
                    xtv[r, pl.ds(16 * v, 16)] = d * e
                    ov[r, pl.ds(16 * v, 16)] = w0 * e
            return _
        lax.fori_loop(0, CB // 16, row_grp, None)
        pltpu.sync_copy(xtv, xt0_o.at[pl.ds(gb, CB)])
        pltpu.sync_copy(ov, out0_o.at[pl.ds(gb, CB)])
        return _
    lax.fori_loop(0, TR // CB, x_body, None)


def _make_layer(widx: int, last: bool):
    outs = [jax.ShapeDtypeStruct((NPAD, D), _f32)]
    if not last:
        outs.append(jax.ShapeDtypeStruct((NPAD, D), _f32))

    @functools.partial(
        pl.kernel,
        out_type=tuple(outs),
        mesh=_MESH,
        compiler_params=_CP,
        scratch_types=(
            pltpu.VMEM_SHARED((NHALF, D), _f32),
            pltpu.VMEM((EB,), _i32),
            pltpu.VMEM((EB,), _i32),
            pltpu.VMEM((CB, D), _f32),
            pltpu.VMEM((CB, D), _f32),
            pltpu.VMEM((CB, D), _f32),
            pltpu.VMEM((CB,), _f32),
            pltpu.VMEM((16,), _f32),
        ),
    )
    def _layer(row_hbm, col_hbm, xt_in, out_in, dinv_hbm, w_hbm,
               out_o, *rest):
        if last:
            (acc, rowB, clB, msg, outv, xtv, dv, wv) = rest
            xt_o = None
        else:
            (xt_o, acc, rowB, clB, msg, outv, xtv, dv, wv) = rest
        c = lax.axis_index("c")
        s = lax.axis_index("s")
        sc_base = c * NHALF

        _zero_rows(msg, CB)

        def z_body(i, _):
            pltpu.sync_copy(msg, acc.at[pl.ds(s * TR + i * CB, CB)])
            return _
        lax.fori_loop(0, TR // CB, z_body, None)
        pltpu.sync_copy(w_hbm, wv)

        plsc.subcore_barrier()

        def blk_body(b, _):
            e0 = s * EPT + b * EB
            pltpu.sync_copy(row_hbm.at[pl.ds(e0, EB)], rowB)
            pltpu.sync_copy(col_hbm.at[pl.ds(e0, EB)], clB)

            def cl_body(i, _):
                v = clB[pl.ds(16 * i, 16)]
                lc = v - sc_base
                lc = jnp.where((lc >= 0) & (lc < NHALF), lc, -1)
                clB[pl.ds(16 * i, 16)] = lc
                return _
            lax.fori_loop(0, EB // 16, cl_body, None)

            def e_body(j, _):
                pltpu.sync_copy(xt_in.at[rowB.at[pl.ds(j * CB, CB)]], msg)
                pltpu.sync_copy(
                    msg,
                    acc.at[plsc.Indices(clB.at[pl.ds(j * CB, CB)],
                                        ignored_value=-1)],
                    add=True,
                )
                return _
            lax.fori_loop(0, NCB, e_body, None)
            return _
        lax.fori_loop(0, NEB, blk_body, None)
        plsc.subcore_barrier()

        g0 = sc_base + s * TR
        w = wv[pl.ds(0, 16)][widx]

        def wb_body(ch, _):
            gb = g0 + ch * CB
            lb = s * TR + ch * CB
            pltpu.sync_copy(acc.at[pl.ds(lb, CB)], msg)
            pltpu.sync_copy(dinv_hbm.at[pl.ds(gb, CB)], dv)
            pltpu.sync_copy(out_in.at[pl.ds(gb, CB)], outv)

            def row_grp(g, _):
                dvec = dv[pl.ds(16 * g, 16)]
                for k in range(16):
                    d = dvec[k]
                    r = 16 * g + k
                    for v in range(D // 16):
                        a = msg[r, pl.ds(16 * v, 16)]
                        xn = d * a
                        outv[r, pl.ds(16 * v, 16)] = (
                            outv[r, pl.ds(16 * v, 16)] + w * xn
                        )
                        if not last:
                            xtv[r, pl.ds(16 * v, 16)] = d * xn
                return _
            lax.fori_loop(0, CB // 16, row_grp, None)
            pltpu.sync_copy(outv, out_o.at[pl.ds(gb, CB)])
            if not last:
                pltpu.sync_copy(xtv, xt_o.at[pl.ds(gb, CB)])
            return _
        lax.fori_loop(0, TR // CB, wb_body, None)

    return _layer


@functools.partial(
    pl.kernel,
    out_type=jax.ShapeDtypeStruct((NLP,), _f32),
    mesh=_MESH,
    compiler_params=_CP,
    scratch_types=(
        pltpu.VMEM((LCB,), _i32),
        pltpu.VMEM((LCB,), _i32),
        pltpu.VMEM((LCB, D), _f32),
        pltpu.VMEM((LCB, D), _f32),
        pltpu.VMEM((LCB,), _f32),
    ),
)
def _k4(out_hbm, lsrc, ldst, res_o, siv, div_, av, bv, rv):
    c = lax.axis_index("c")
    s = lax.axis_index("s")
    wid = s * NC + c
    base0 = wid * LPT

    def ch_body(ch, _):
        eb = base0 + ch * LCB
        pltpu.sync_copy(lsrc.at[pl.ds(eb, LCB)], siv)
        pltpu.sync_copy(ldst.at[pl.ds(eb, LCB)], div_)
        pltpu.sync_copy(out_hbm.at[siv], av)
        pltpu.sync_copy(out_hbm.at[div_], bv)

        lane = lax.iota(_i32, 16)

        def row_grp(g, _):
            t = jnp.zeros((16,), _f32)
            for k in range(16):
                r = 16 * g + k
                acc = av[r, pl.ds(0, 16)] * bv[r, pl.ds(0, 16)]
                for v in range(1, D // 16):
                    acc = acc + (av[r, pl.ds(16 * v, 16)]
                                 * bv[r, pl.ds(16 * v, 16)])
                t = jnp.where(lane == k, jnp.sum(acc), t)
            rv[pl.ds(16 * g, 16)] = t
            return _
        lax.fori_loop(0, LCB // 16, row_grp, None)
        pltpu.sync_copy(rv, res_o.at[pl.ds(eb, LCB)])
        return _
    lax.fori_loop(0, LNCH, ch_body, None)


_LAYER1 = _make_layer(1, last=False)
_LAYER2 = _make_layer(2, last=False)
_LAYER3 = _make_layer(3, last=True)


def kernel(edge_index, edge_label_index, embedding, alpha):
    row = edge_index[0]
    col = edge_index[1]
    w = jax.nn.softmax(alpha, axis=-1)
    w16 = jnp.zeros((16,), _f32).at[:4].set(w)
    lsrc = jnp.zeros((NLP,), _i32).at[:NL].set(edge_label_index[0])
    ldst = jnp.zeros((NLP,), _i32).at[:NL].set(edge_label_index[1])

    dinv, xt0, out0 = _k0(col, embedding, w16)
    out1, xt1 = _LAYER1(row, col, xt0, out0, dinv, w16)
    out2, xt2 = _LAYER2(row, col, xt1, out1, dinv, w16)
    (out3,) = _LAYER3(row, col, xt2, out2, dinv, w16)
    res = _k4(out3, lsrc, ldst)
    return res[:NL]


# speedup vs baseline: 14.0307x; 1.5519x over previous
import functools

import jax
import jax.numpy as jnp
from jax import lax
from jax.experimental import pallas as pl
from jax.experimental.pallas import tpu as pltpu
from jax.experimental.pallas import tpu_sc as plsc

N = 50000
D = 64
E = 800000
NL = 100000
NC, NS = 2, 16

NHALF = 25600
NPAD = 2 * NHALF
TR = NHALF // NS
EPT = E // NS
CB = 80
EB = 2000
NEB = EPT // EB
NCB = EB // CB
NLP = 102400
LPT = NLP // (NC * NS)
LCB = 128
LNCH = LPT // LCB

_MESH = plsc.VectorSubcoreMesh(
    core_axis_name="c", subcore_axis_name="s", num_cores=NC, num_subcores=NS
)

_f32 = jnp.float32
_i32 = jnp.int32

_CP = pltpu.CompilerParams(
    use_tc_tiling_on_sc=False, needs_layout_passes=False
)


def _rsqrt16(x):
    s = jnp.maximum(x, 1.0)
    for _ in range(18):
        s = 0.5 * (s + x / s)
    return jnp.where(x > 0.0, 1.0 / s, 0.0)


def _zero_rows(buf, rows):
    z = jnp.zeros((16,), _f32)

    def body(r, _):
        for v in range(D // 16):
            buf[r, pl.ds(16 * v, 16)] = z
        return _

    lax.fori_loop(0, rows, body, None)


@functools.partial(
    pl.kernel,
    out_type=(
        jax.ShapeDtypeStruct((NPAD,), _f32),
        jax.ShapeDtypeStruct((NPAD, D), _f32),
        jax.ShapeDtypeStruct((NPAD, D), _f32),
    ),
    mesh=_MESH,
    compiler_params=_CP,
    scratch_types=(
        pltpu.VMEM_SHARED((NPAD,), _f32),
        pltpu.VMEM((NPAD // NS,), _f32),
        pltpu.VMEM((CB,), _f32),
        pltpu.VMEM((10000,), _i32),
        pltpu.VMEM((NPAD // NS,), _f32),
        pltpu.VMEM((CB,), _f32),
        pltpu.VMEM((CB, D), _f32),
        pltpu.VMEM((CB, D), _f32),
        pltpu.VMEM((CB, D), _f32),
        pltpu.VMEM((16,), _f32),
    ),
)
def _k0(col_hbm, emb_hbm, w_hbm, dinv_o, xt0_o, out0_o,
        deg_sh, zb, ones, colv, degv, dv, embv, xtv, ov, wv):
    c = lax.axis_index("c")
    s = lax.axis_index("s")
    spt = NPAD // NS

    def zb_body(i, _):
        zb[pl.ds(16 * i, 16)] = jnp.zeros((16,), _f32)
        return _
    lax.fori_loop(0, spt // 16, zb_body, None)
    pltpu.sync_copy(zb.at[pl.ds(0, spt)], deg_sh.at[pl.ds(s * spt, spt)])
    for k in range(CB // 16):
        ones[pl.ds(16 * k, 16)] = jnp.ones((16,), _f32)
    pltpu.sync_copy(w_hbm, wv)
    plsc.subcore_barrier()

    for b in range(5):
        pltpu.sync_copy(col_hbm.at[pl.ds(s * EPT + b * 10000, 10000)], colv)

        def deg_body(j, _):
            pltpu.sync_copy(
                ones, deg_sh.at[colv.at[pl.ds(j * CB, CB)]], add=True
            )
            return _
        lax.fori_loop(0, 10000 // CB, deg_body, None)
    plsc.subcore_barrier()

    @pl.when(c == 0)
    def _():
        pltpu.sync_copy(deg_sh.at[pl.ds(s * spt, spt)], degv)

        def nwt(i, _):
            degv[pl.ds(16 * i, 16)] = _rsqrt16(degv[pl.ds(16 * i, 16)])
            return _
        lax.fori_loop(0, spt // 16, nwt, None)
        pltpu.sync_copy(degv, dinv_o.at[pl.ds(s * spt, spt)])

    g0 = c * NHALF + s * TR

    def x_body(ch, _):
        gb = g0 + ch * CB
        eb = jnp.minimum(gb, N - CB)
        pltpu.sync_copy(emb_hbm.at[pl.ds(eb, CB)], embv)
        pltpu.sync_copy(deg_sh.at[pl.ds(gb, CB)], dv)
        for k in range(CB // 16):
            dv[pl.ds(16 * k, 16)] = _rsqrt16(dv[pl.ds(16 * k, 16)])
        w0 = wv[pl.ds(0, 16)][0]

        def row_grp(g, _):
            dvec = dv[pl.ds(16 * g, 16)]
            for k in range(16):
                d = dvec[k]
                r = 16 * g + k
                for v in range(D // 16):
                    e = embv[r, pl.ds(16 * v, 16)]
                    xtv[r, pl.ds(16 * v, 16)] = d * e
                    ov[r, pl.ds(16 * v, 16)] = w0 * e
            return _
        lax.fori_loop(0, CB // 16, row_grp, None)
        pltpu.sync_copy(xtv, xt0_o.at[pl.ds(gb, CB)])
        pltpu.sync_copy(ov, out0_o.at[pl.ds(gb, CB)])
        return _
    lax.fori_loop(0, TR // CB, x_body, None)


def _make_layer(widx: int, last: bool):
    outs = [jax.ShapeDtypeStruct((NPAD, D), _f32)]
    if not last:
        outs.append(jax.ShapeDtypeStruct((NPAD, D), _f32))

    @functools.partial(
        pl.kernel,
        out_type=tuple(outs),
        mesh=_MESH,
        compiler_params=_CP,
        scratch_types=(
            pltpu.VMEM_SHARED((NHALF, D), _f32),
            pltpu.VMEM((EB,), _i32),
            pltpu.VMEM((EB,), _i32),
            pltpu.VMEM((CB, D), _f32),
            pltpu.VMEM((CB, D), _f32),
            pltpu.VMEM((CB, D), _f32),
            pltpu.VMEM((CB, D), _f32),
            pltpu.VMEM((CB,), _f32),
            pltpu.VMEM((16,), _f32),
            pltpu.SemaphoreType.DMA,
            pltpu.SemaphoreType.DMA,
        ),
    )
    def _layer(row_hbm, col_hbm, xt_in, out_in, dinv_hbm, w_hbm,
               out_o, *rest):
        if last:
            (acc, rowB, clB, msg, msg2, outv, xtv, dv, wv,
             semA, semB) = rest
            xt_o = None
        else:
            (xt_o, acc, rowB, clB, msg, msg2, outv, xtv, dv, wv,
             semA, semB) = rest
        c = lax.axis_index("c")
        s = lax.axis_index("s")
        sc_base = c * NHALF

        _zero_rows(msg, CB)

        def z_body(i, _):
            pltpu.sync_copy(msg, acc.at[pl.ds(s * TR + i * CB, CB)])
            return _
        lax.fori_loop(0, TR // CB, z_body, None)
        pltpu.sync_copy(w_hbm, wv)

        plsc.subcore_barrier()

        def fire(j, buf, sem):
            return pltpu.async_copy(
                xt_in.at[rowB.at[pl.ds(j * CB, CB)]], buf, sem
            )

        def drain(j, buf, sem):
            pltpu.make_async_copy(
                xt_in.at[rowB.at[pl.ds(j * CB, CB)]], buf, sem
            ).wait()

        def transform(j):
            for k in range(CB // 16):
                v = clB[pl.ds(j * CB + 16 * k, 16)]
                lc = v - sc_base
                lc = jnp.where((lc >= 0) & (lc < NHALF), lc, -1)
                clB[pl.ds(j * CB + 16 * k, 16)] = lc

        def scat(j, buf):
            pltpu.sync_copy(
                buf,
                acc.at[plsc.Indices(clB.at[pl.ds(j * CB, CB)],
                                    ignored_value=-1)],
                add=True,
            )

        def blk_body(b, _):
            e0 = s * EPT + b * EB
            pltpu.sync_copy(row_hbm.at[pl.ds(e0, EB)], rowB)
            pltpu.sync_copy(col_hbm.at[pl.ds(e0, EB)], clB)
            fire(0, msg, semA)

            def pair_body(i, _):
                j = 2 * i
                fire(j + 1, msg2, semB)
                transform(j)
                drain(j, msg, semA)
                scat(j, msg)
                fire(j + 2, msg, semA)
                transform(j + 1)
                drain(j + 1, msg2, semB)
                scat(j + 1, msg2)
                return _
            lax.fori_loop(0, (NCB - 1) // 2, pair_body, None)
            transform(NCB - 1)
            drain(NCB - 1, msg, semA)
            scat(NCB - 1, msg)
            return _
        lax.fori_loop(0, NEB, blk_body, None)
        plsc.subcore_barrier()

        g0 = sc_base + s * TR
        w = wv[pl.ds(0, 16)][widx]

        def wb_body(ch, _):
            gb = g0 + ch * CB
            lb = s * TR + ch * CB
            pltpu.sync_copy(acc.at[pl.ds(lb, CB)], msg)
            pltpu.sync_copy(dinv_hbm.at[pl.ds(gb, CB)], dv)
            pltpu.sync_copy(out_in.at[pl.ds(gb, CB)], outv)

            def row_grp(g, _):
                dvec = dv[pl.ds(16 * g, 16)]
                for k in range(16):
                    d = dvec[k]
                    r = 16 * g + k
                    for v in range(D // 16):
                        a = msg[r, pl.ds(16 * v, 16)]
                        xn = d * a
                        outv[r, pl.ds(16 * v, 16)] = (
                            outv[r, pl.ds(16 * v, 16)] + w * xn
                        )
                        if not last:
                            xtv[r, pl.ds(16 * v, 16)] = d * xn
                return _
            lax.fori_loop(0, CB // 16, row_grp, None)
            pltpu.sync_copy(outv, out_o.at[pl.ds(gb, CB)])
            if not last:
                pltpu.sync_copy(xtv, xt_o.at[pl.ds(gb, CB)])
            return _
        lax.fori_loop(0, TR // CB, wb_body, None)

    return _layer


@functools.partial(
    pl.kernel,
    out_type=jax.ShapeDtypeStruct((NLP,), _f32),
    mesh=_MESH,
    compiler_params=_CP,
    scratch_types=(
        pltpu.VMEM((LCB,), _i32),
        pltpu.VMEM((LCB,), _i32),
        pltpu.VMEM((LCB, D), _f32),
        pltpu.VMEM((LCB, D), _f32),
        pltpu.VMEM((LCB,), _f32),
    ),
)
def _k4(out_hbm, lsrc, ldst, res_o, siv, div_, av, bv, rv):
    c = lax.axis_index("c")
    s = lax.axis_index("s")
    wid = s * NC + c
    base0 = wid * LPT

    def ch_body(ch, _):
        eb = base0 + ch * LCB
        pltpu.sync_copy(lsrc.at[pl.ds(eb, LCB)], siv)
        pltpu.sync_copy(ldst.at[pl.ds(eb, LCB)], div_)
        pltpu.sync_copy(out_hbm.at[siv], av)
        pltpu.sync_copy(out_hbm.at[div_], bv)

        lane = lax.iota(_i32, 16)

        def row_grp(g, _):
            t = jnp.zeros((16,), _f32)
            for k in range(16):
                r = 16 * g + k
                acc = av[r, pl.ds(0, 16)] * bv[r, pl.ds(0, 16)]
                for v in range(1, D // 16):
                    acc = acc + (av[r, pl.ds(16 * v, 16)]
                                 * bv[r, pl.ds(16 * v, 16)])
                t = jnp.where(lane == k, jnp.sum(acc), t)
            rv[pl.ds(16 * g, 16)] = t
            return _
        lax.fori_loop(0, LCB // 16, row_grp, None)
        pltpu.sync_copy(rv, res_o.at[pl.ds(eb, LCB)])
        return _
    lax.fori_loop(0, LNCH, ch_body, None)


_LAYER1 = _make_layer(1, last=False)
_LAYER2 = _make_layer(2, last=False)
_LAYER3 = _make_layer(3, last=True)


def kernel(edge_index, edge_label_index, embedding, alpha):
    row = edge_index[0]
    col = edge_index[1]
    w = jax.nn.softmax(alpha, axis=-1)
    w16 = jnp.zeros((16,), _f32).at[:4].set(w)
    lsrc = jnp.zeros((NLP,), _i32).at[:NL].set(edge_label_index[0])
    ldst = jnp.zeros((NLP,), _i32).at[:NL].set(edge_label_index[1])

    dinv, xt0, out0 = _k0(col, embedding, w16)
    out1, xt1 = _LAYER1(row, col, xt0, out0, dinv, w16)
    out2, xt2 = _LAYER2(row, col, xt1, out1, dinv, w16)
    (out3,) = _LAYER3(row, col, xt2, out2, dinv, w16)
    res = _k4(out3, lsrc, ldst)
    return res[:NL]
